# baseline (speedup 1.0000x reference)
import functools

import jax
import jax.numpy as jnp
from jax import lax
from jax.experimental import pallas as pl
from jax.experimental.pallas import tpu as pltpu
from jax.experimental.pallas import tpu_sc as plsc

_LANES = 16
_CHUNK = 160
_MAGIC = 26215


def _scan_kernel(B, V, D, NW):
    mesh = plsc.VectorSubcoreMesh(core_axis_name="c", subcore_axis_name="s")
    n_chunks = V // _CHUNK
    max_local = (n_chunks + NW - 1) // NW
    n_groups = B // _LANES
    idx_rows = B // 128

    @functools.partial(
        pl.kernel,
        mesh=mesh,
        out_type=jax.ShapeDtypeStruct((B + 8, 2 * D), jnp.float32),
        scratch_types=[
            pltpu.VMEM((idx_rows, 128), jnp.int32),
            pltpu.VMEM((B + 16,), jnp.int32),
            pltpu.VMEM((B + 16,), jnp.int32),
            pltpu.VMEM((144,), jnp.int32),
            pltpu.VMEM((_CHUNK, D), jnp.float32),
            pltpu.VMEM((_CHUNK, D), jnp.float32),
            pltpu.VMEM((256, 2 * D), jnp.float32),
            pltpu.VMEM((2, 128), jnp.int32),
            pltpu.SemaphoreType.DMA,
            pltpu.SemaphoreType.DMA,
            pltpu.SemaphoreType.DMA,
        ],
        compiler_params=pltpu.CompilerParams(needs_layout_passes=False),
    )
    def k(
        idx_hbm,
        table_hbm,
        out_hbm,
        idx_v,
        mlist,
        rlist,
        plist,
        cbuf0,
        cbuf1,
        stage,
        prow,
        srd0,
        srd1,
        ssc,
    ):
        nc = plsc.get_sparse_core_info().num_cores
        wid = lax.axis_index("s") * nc + lax.axis_index("c")
        pltpu.sync_copy(idx_hbm, idx_v)
        lanes = lax.iota(jnp.int32, _LANES)
        cbufs = (cbuf0, cbuf1)
        srds = (srd0, srd1)

        n_my = (n_chunks - wid + NW - 1) // NW

        def chunk_of(r16):
            return lax.shift_right_logical(
                lax.shift_right_logical(r16, 5) * _MAGIC, 17
            )

        def match_group(g, ptr):
            row = g // 8
            col = (g % 8) * _LANES
            r16 = idx_v[row, pl.ds(col, _LANES)]
            mine = (chunk_of(r16) & (NW - 1)) == wid
            pos16 = g * _LANES + lanes
            keys = jnp.where(mine, lanes, _LANES)
            _, vs = plsc.sort_key_val(keys, pos16)
            _, rss = plsc.sort_key_val(keys, r16)
            mlist[pl.ds(ptr, _LANES)] = vs
            rlist[pl.ds(ptr, _LANES)] = rss
            cnt = plsc.all_reduce_population_count(mine)
            return ptr + cnt[0]

        mcnt = lax.fori_loop(0, n_groups, match_group, jnp.int32(0))
        mgroups = lax.shift_right_logical(mcnt + _LANES - 1, 4)

        def process_chunk(c_l, ap, buf):
            c = jnp.minimum(c_l * NW + wid, n_chunks - 1)
            lo = c * _CHUNK

            def mgroup_body(q, ap):
                rvec = rlist[pl.ds(q * _LANES, _LANES)]
                valid = (q * _LANES + lanes) < mcnt
                hit = valid & (chunk_of(rvec) == c)
                cntv = plsc.all_reduce_population_count(hit)
                cnt = cntv[0]
                ap_new = ap + cnt
                fb = lax.shift_right_logical(ap, 7)
                crossing = lax.shift_right_logical(ap_new, 7) > fb

                @pl.when(crossing & (fb >= 1))
                def _wait_prev():
                    pltpu.make_async_copy(
                        stage.at[pl.ds(0, 128)],
                        out_hbm.at[prow.at[(fb - 1) & 1]],
                        ssc,
                    ).wait()

                @pl.when(cnt > 0)
                def _emit():
                    raw = mlist[pl.ds(q * _LANES, _LANES)]
                    pos16 = jnp.where(valid, raw, 0)
                    keys = jnp.where(hit, lanes, _LANES)
                    _, vs = plsc.sort_key_val(keys, pos16)
                    _, rss = plsc.sort_key_val(keys, rvec)
                    plist[pl.ds(ap & 127, _LANES)] = vs
                    for j in range(_LANES):

                        @pl.when(j < cnt)
                        def _copy_row(j=j):
                            rowin = rss[j] - lo
                            slot = (ap + j) & 255
                            for kk in range(D // _LANES):
                                stage[slot, pl.ds(kk * _LANES, _LANES)] = buf[
                                    rowin, pl.ds(kk * _LANES, _LANES)
                                ]

                @pl.when(crossing)
                def _issue():
                    for kq in range(8):
                        prow[fb & 1, pl.ds(kq * _LANES, _LANES)] = plist[
                            pl.ds(kq * _LANES, _LANES)
                        ]
                    spill = plist[pl.ds(128, _LANES)]
                    plist[pl.ds(0, _LANES)] = spill
                    pltpu.async_copy(
                        stage.at[pl.ds((fb & 1) * 128, 128)],
                        out_hbm.at[prow.at[fb & 1]],
                        ssc,
                    )

                return ap_new

            mg = jnp.where(c_l < n_my, mgroups, 0)
            return lax.fori_loop(0, mg, mgroup_body, ap)

        def read_chunk(c_l, b):
            c = jnp.minimum(c_l * NW + wid, n_chunks - 1)
            pltpu.async_copy(
                table_hbm.at[pl.ds(c * _CHUNK, _CHUNK)], cbufs[b], srds[b]
            )

        def wait_chunk(c_l, b):
            c = jnp.minimum(c_l * NW + wid, n_chunks - 1)
            pltpu.make_async_copy(
                table_hbm.at[pl.ds(c * _CHUNK, _CHUNK)], cbufs[b], srds[b]
            ).wait()

        n_slots = 2 * ((max_local + 1) // 2)
        read_chunk(jnp.int32(0), 0)

        def pair_body(g, ap):
            for b in range(2):
                c_l = 2 * g + b
                wait_chunk(c_l, b)

                @pl.when(c_l + 1 < n_slots)
                def _prefetch(c_l=c_l, b=b):
                    read_chunk(c_l + 1, 1 - b)

                ap = process_chunk(c_l, ap, cbufs[b])
            return ap

        ap = lax.fori_loop(0, n_slots // 2, pair_body, jnp.int32(0))

        rem = ap & 127
        fb_full = lax.shift_right_logical(ap, 7)

        @pl.when(rem > 0)
        def _tail():
            for kq in range(8):
                lanepos = kq * _LANES + lanes
                veck = plist[pl.ds(kq * _LANES, _LANES)]
                prow[fb_full & 1, pl.ds(kq * _LANES, _LANES)] = jnp.where(
                    lanepos < rem, veck, B
                )

            @pl.when(fb_full >= 1)
            def _wait_prev():
                pltpu.make_async_copy(
                    stage.at[pl.ds(0, 128)],
                    out_hbm.at[prow.at[(fb_full - 1) & 1]],
                    ssc,
                ).wait()

            pltpu.async_copy(
                stage.at[pl.ds((fb_full & 1) * 128, 128)],
                out_hbm.at[prow.at[fb_full & 1]],
                ssc,
            )

        outstanding = jnp.where(fb_full + jnp.where(rem > 0, 1, 0) > 0, 1, 0)

        def drain(i, carry):
            pltpu.make_async_copy(
                stage.at[pl.ds(0, 128)],
                out_hbm.at[prow.at[0]],
                ssc,
            ).wait()
            return carry

        lax.fori_loop(0, outstanding, drain, 0)

    return k


def kernel(indices, autodecoder_embeddings):
    (B,) = indices.shape
    V, D = autodecoder_embeddings.shape
    info = plsc.get_sparse_core_info()
    NC, NS = info.num_cores, info.num_subcores
    NW = NC * NS
    idx2d = indices.astype(jnp.int32).reshape(B // 128, 128)
    out = _scan_kernel(B, V, D, NW)(idx2d, autodecoder_embeddings)
    return out[:B, :D]


# speedup vs baseline: 2.6630x; 2.6630x over previous
import functools

import jax
import jax.numpy as jnp
from jax import lax
from jax.experimental import pallas as pl
from jax.experimental.pallas import tpu as pltpu
from jax.experimental.pallas import tpu_sc as plsc

_CHUNK_IDX = 128
_LANES = 16


def _gather_kernel(B, V, D, NW, b_per_w, n_idx_chunks):
    mesh = plsc.VectorSubcoreMesh(core_axis_name="c", subcore_axis_name="s")
    n_groups = b_per_w // _LANES
    half = V // 2

    @functools.partial(
        pl.kernel,
        mesh=mesh,
        out_type=jax.ShapeDtypeStruct((B, D), jnp.float32),
        scratch_types=[
            pltpu.VMEM((n_idx_chunks, _CHUNK_IDX), jnp.int32),
            pltpu.VMEM((n_idx_chunks, _CHUNK_IDX), jnp.int32),
            pltpu.VMEM((_CHUNK_IDX, 2 * D), jnp.float32),
            pltpu.VMEM((_CHUNK_IDX, D), jnp.float32),
            pltpu.SemaphoreType.DMA,
        ],
    )
    def k(idx_hbm, lin_hbm, out_hbm, idx_v, p_v, rows_v, out_v, sem):
        nc = plsc.get_sparse_core_info().num_cores
        wid = lax.axis_index("s") * nc + lax.axis_index("c")
        row_base = wid * n_idx_chunks
        pltpu.sync_copy(idx_hbm.at[pl.ds(row_base, n_idx_chunks)], idx_v)

        per_row = _CHUNK_IDX // _LANES

        def shift_group(g, carry):
            r = g // per_row
            col = (g % per_row) * _LANES
            vec = idx_v[r, pl.ds(col, _LANES)]
            p_v[r, pl.ds(col, _LANES)] = jnp.where(vec >= half, vec - half, vec)
            return carry

        lax.fori_loop(0, n_groups, shift_group, 0)

        groups_per_chunk = _CHUNK_IDX // _LANES
        for hh in range(n_idx_chunks):
            pltpu.async_copy(lin_hbm.at[p_v.at[hh]], rows_v, sem).wait()

            def select_group(g, carry, hh=hh):
                col = g * _LANES
                vec = idx_v[hh, pl.ds(col, _LANES)]
                for j in range(_LANES):
                    i = g * _LANES + j
                    src = jnp.where(vec[j] >= half, D, 0)
                    for kk in range(D // _LANES):
                        out_v[i, pl.ds(kk * _LANES, _LANES)] = rows_v[
                            i, pl.ds(src + kk * _LANES, _LANES)
                        ]
                return carry

            lax.fori_loop(0, groups_per_chunk, select_group, 0)
            pltpu.sync_copy(
                out_v,
                out_hbm.at[pl.ds(wid * b_per_w + hh * _CHUNK_IDX, _CHUNK_IDX)],
            )

    return k


def kernel(indices, autodecoder_embeddings):
    (B,) = indices.shape
    V, D = autodecoder_embeddings.shape
    info = plsc.get_sparse_core_info()
    NC, NS = info.num_cores, info.num_subcores
    NW = NC * NS
    b_per_w = B // NW
    n_idx_chunks = b_per_w // _CHUNK_IDX
    idx2d = indices.astype(jnp.int32).reshape(NW * n_idx_chunks, _CHUNK_IDX)
    half = V // 2
    lin = jnp.concatenate(
        [autodecoder_embeddings[:half], autodecoder_embeddings[half:]], axis=1
    )
    return _gather_kernel(B, V, D, NW, b_per_w, n_idx_chunks)(idx2d, lin)


# speedup vs baseline: 5.5779x; 2.0946x over previous
import functools

import jax
import jax.numpy as jnp
from jax import lax
from jax.experimental import pallas as pl
from jax.experimental.pallas import tpu as pltpu
from jax.experimental.pallas import tpu_sc as plsc

_LANES = 16


def _gather_kernel(B, D, NC, NW, b_per_w):
    mesh = plsc.VectorSubcoreMesh(core_axis_name="c", subcore_axis_name="s")
    n_groups = b_per_w // _LANES

    @functools.partial(
        pl.kernel,
        mesh=mesh,
        out_type=jax.ShapeDtypeStruct((B, D), jnp.float32),
        scratch_types=[
            pltpu.VMEM((b_per_w,), jnp.int32),
            pltpu.VMEM((b_per_w, D), jnp.float32),
            pltpu.SemaphoreType.DMA,
            pltpu.SemaphoreType.DMA,
            pltpu.SemaphoreType.DMA,
            pltpu.SemaphoreType.DMA,
        ],
    )
    def k(idx_hbm, table_hbm, out_hbm, idx_v, rows_v, s0, s1, s2, s3):
        wid = lax.axis_index("s") * NC + lax.axis_index("c")
        sems = (s0, s1, s2, s3)
        pltpu.sync_copy(idx_hbm.at[wid], idx_v)

        def group(g, carry):
            vec = idx_v[pl.ds(g * _LANES, _LANES)]
            for j in range(_LANES):
                r = vec[j]
                pltpu.async_copy(
                    table_hbm.at[r], rows_v.at[g * _LANES + j], sems[j % 4]
                )
            return carry

        lax.fori_loop(0, n_groups, group, 0)
        q = b_per_w // 4
        for i, s in enumerate(sems):
            pltpu.make_async_copy(
                table_hbm.at[pl.ds(0, q)], rows_v.at[pl.ds(i * q, q)], s
            ).wait()
        pltpu.sync_copy(rows_v, out_hbm.at[pl.ds(wid * b_per_w, b_per_w)])

    return k


def kernel(indices, autodecoder_embeddings):
    (B,) = indices.shape
    V, D = autodecoder_embeddings.shape
    info = plsc.get_sparse_core_info()
    NC, NS = info.num_cores, info.num_subcores
    NW = NC * NS
    b_per_w = B // NW
    idx2d = indices.astype(jnp.int32).reshape(NW, b_per_w)
    k = _gather_kernel(B, D, NC, NW, b_per_w)
    return k(idx2d, autodecoder_embeddings)
